# Initial kernel scaffold; baseline (speedup 1.0000x reference)
import jax
import jax.numpy as jnp
from jax.experimental import pallas as pl


def kernel(inputs, edge_index, W1, b1, W2, b2):
    raise NotImplementedError("write your pallas kernel here")


# speedup vs baseline: 10.4455x; 10.4455x over previous
import functools

import jax
import jax.numpy as jnp
from jax import lax
from jax.experimental import pallas as pl
from jax.experimental.pallas import tpu as pltpu
from jax.experimental.pallas import tpu_sc as plsc

N = 10000
E = 320000
F_IN = 128
HID = 128
NCLS = 64

NC = 2
NS = 16
NW = NC * NS

CHUNK = 128
CH = 80
IDXW = 40
EPAD = NW * CH * CHUNK
NPAD = 10240
RPT = NPAD // NS
PAD_SPREAD = NPAD - N

_mesh = plsc.VectorSubcoreMesh(core_axis_name="c", subcore_axis_name="s")


@functools.partial(
    pl.kernel,
    out_type=(
        jax.ShapeDtypeStruct((NC, NPAD, 16), jnp.float32),
        jax.ShapeDtypeStruct((NC, NPAD, 16), jnp.float32),
    ),
    mesh=_mesh,
    compiler_params=pltpu.CompilerParams(use_tc_tiling_on_sc=False),
    scratch_types=[
        pltpu.VMEM((CH, CHUNK), jnp.int32),
        pltpu.VMEM((CH, CHUNK), jnp.int32),
        pltpu.VMEM((CHUNK, 16), jnp.float32),
        pltpu.VMEM_SHARED((NPAD, 16), jnp.float32),
        pltpu.VMEM_SHARED((NPAD, 16), jnp.float32),
    ],
)
def _deg_kernel(src_hbm, dst_hbm, ones_hbm, zeros_hbm, od_out, id_out,
                sidx, didx, ones_v, od_sh, id_sh):
    c = lax.axis_index("c")
    s = lax.axis_index("s")
    wid = c * NS + s
    pltpu.sync_copy(src_hbm.at[wid], sidx)
    pltpu.sync_copy(dst_hbm.at[wid], didx)
    pltpu.sync_copy(ones_hbm, ones_v)
    pltpu.sync_copy(zeros_hbm, od_sh.at[pl.ds(s * RPT, RPT)])
    pltpu.sync_copy(zeros_hbm, id_sh.at[pl.ds(s * RPT, RPT)])
    plsc.subcore_barrier()

    def body(j, carry):
        pltpu.sync_copy(ones_v, od_sh.at[sidx.at[j]], add=True)
        pltpu.sync_copy(ones_v, id_sh.at[didx.at[j]], add=True)
        return carry

    lax.fori_loop(0, CH, body, 0)
    plsc.subcore_barrier()
    pltpu.sync_copy(od_sh.at[pl.ds(s * RPT, RPT)],
                    od_out.at[c, pl.ds(s * RPT, RPT)])
    pltpu.sync_copy(id_sh.at[pl.ds(s * RPT, RPT)],
                    id_out.at[c, pl.ds(s * RPT, RPT)])


def _make_mp(F):
    params = None if F % 128 == 0 else pltpu.CompilerParams(
        use_tc_tiling_on_sc=False)

    @functools.partial(
        pl.kernel,
        out_type=jax.ShapeDtypeStruct((NC, NPAD, F), jnp.float32),
        mesh=_mesh,
        compiler_params=params,
        scratch_types=[
            pltpu.VMEM((IDXW, CHUNK), jnp.int32),
            pltpu.VMEM((IDXW, CHUNK), jnp.int32),
            pltpu.VMEM((2, CHUNK, F), jnp.float32),
            pltpu.VMEM_SHARED((NPAD, F), jnp.float32),
            pltpu.SemaphoreType.DMA,
            pltpu.SemaphoreType.DMA,
        ],
    )
    def _mp(h_hbm, src_hbm, dst_hbm, zeros_hbm, out_hbm,
            sidx, didx, gbuf, agg_sh, sem0, sem1):
        c = lax.axis_index("c")
        s = lax.axis_index("s")
        wid = c * NS + s
        pltpu.sync_copy(zeros_hbm, agg_sh.at[pl.ds(s * RPT, RPT)])
        plsc.subcore_barrier()

        def body(i, carry):
            j0 = 2 * i
            j1 = j0 + 1
            pltpu.async_copy(h_hbm.at[sidx.at[j1]], gbuf.at[1], sem1)
            pltpu.make_async_copy(h_hbm.at[sidx.at[j0]], gbuf.at[0],
                                  sem0).wait()
            pltpu.sync_copy(gbuf.at[0], agg_sh.at[didx.at[j0]], add=True)

            @pl.when(i < IDXW // 2 - 1)
            def _():
                pltpu.async_copy(h_hbm.at[sidx.at[j0 + 2]], gbuf.at[0], sem0)

            pltpu.make_async_copy(h_hbm.at[sidx.at[j1]], gbuf.at[1],
                                  sem1).wait()
            pltpu.sync_copy(gbuf.at[1], agg_sh.at[didx.at[j1]], add=True)
            return carry

        for win in range(CH // IDXW):
            pltpu.sync_copy(src_hbm.at[wid, pl.ds(win * IDXW, IDXW)], sidx)
            pltpu.sync_copy(dst_hbm.at[wid, pl.ds(win * IDXW, IDXW)], didx)
            pltpu.async_copy(h_hbm.at[sidx.at[0]], gbuf.at[0], sem0)
            lax.fori_loop(0, IDXW // 2, body, 0)

        plsc.subcore_barrier()
        pltpu.sync_copy(agg_sh.at[pl.ds(s * RPT, RPT)],
                        out_hbm.at[c, pl.ds(s * RPT, RPT)])

    return _mp


_mp128 = _make_mp(HID)
_mp64 = _make_mp(NCLS)

BLK = 512


def _inv_sqrt_deg(dcol):
    return jnp.where(dcol > 0, lax.rsqrt(jnp.maximum(dcol, 1.0)), 0.0)


def _l1_kernel(x_ref, w_ref, od_ref, o_ref):
    d = od_ref[0] + od_ref[1]
    sc = _inv_sqrt_deg(d[:, 0:1])
    o_ref[...] = jnp.dot(x_ref[...] * sc, w_ref[...],
                         preferred_element_type=jnp.float32)


def _l2_kernel(agg_ref, id_ref, od_ref, b1_ref, w_ref, o_ref):
    isc = _inv_sqrt_deg((id_ref[0] + id_ref[1])[:, 0:1])
    osc = _inv_sqrt_deg((od_ref[0] + od_ref[1])[:, 0:1])
    h = (agg_ref[0] + agg_ref[1]) * isc + b1_ref[...]
    h = jnp.maximum(h, 0.0) * osc
    o_ref[...] = jnp.dot(h, w_ref[...], preferred_element_type=jnp.float32)


def _out_kernel(agg_ref, id_ref, b2_ref, o_ref):
    isc = _inv_sqrt_deg((id_ref[0] + id_ref[1])[:, 0:1])
    o_ref[...] = (agg_ref[0] + agg_ref[1]) * isc + b2_ref[...]


def kernel(inputs, edge_index, W1, b1, W2, b2):
    src = edge_index[0].astype(jnp.int32)
    dst = edge_index[1].astype(jnp.int32)
    npd = EPAD - E
    pad = N + (jnp.arange(npd, dtype=jnp.int32) % PAD_SPREAD)
    srcr = jnp.concatenate([src, pad]).reshape(NW, CH, CHUNK)
    dstr = jnp.concatenate([dst, pad]).reshape(NW, CH, CHUNK)
    ones16 = jnp.ones((CHUNK, 16), jnp.float32)
    z16 = jnp.zeros((RPT, 16), jnp.float32)
    z128 = jnp.zeros((RPT, HID), jnp.float32)
    z64 = jnp.zeros((RPT, NCLS), jnp.float32)
    xp = jnp.pad(inputs, ((0, NPAD - N), (0, 0)))

    od_p, id_p = _deg_kernel(srcr, dstr, ones16, z16)

    h1s = pl.pallas_call(
        _l1_kernel,
        grid=(NPAD // BLK,),
        in_specs=[
            pl.BlockSpec((BLK, F_IN), lambda i: (i, 0)),
            pl.BlockSpec((F_IN, HID), lambda i: (0, 0)),
            pl.BlockSpec((NC, BLK, 16), lambda i: (0, i, 0)),
        ],
        out_specs=pl.BlockSpec((BLK, HID), lambda i: (i, 0)),
        out_shape=jax.ShapeDtypeStruct((NPAD, HID), jnp.float32),
    )(xp, W1, od_p)

    agg1 = _mp128(h1s, srcr, dstr, z128)

    h2s = pl.pallas_call(
        _l2_kernel,
        grid=(NPAD // BLK,),
        in_specs=[
            pl.BlockSpec((NC, BLK, HID), lambda i: (0, i, 0)),
            pl.BlockSpec((NC, BLK, 16), lambda i: (0, i, 0)),
            pl.BlockSpec((NC, BLK, 16), lambda i: (0, i, 0)),
            pl.BlockSpec((1, HID), lambda i: (0, 0)),
            pl.BlockSpec((HID, NCLS), lambda i: (0, 0)),
        ],
        out_specs=pl.BlockSpec((BLK, NCLS), lambda i: (i, 0)),
        out_shape=jax.ShapeDtypeStruct((NPAD, NCLS), jnp.float32),
    )(agg1, id_p, od_p, b1.reshape(1, HID), W2)

    agg2 = _mp64(h2s, srcr, dstr, z64)

    OBLK = 400
    out = pl.pallas_call(
        _out_kernel,
        grid=(N // OBLK,),
        in_specs=[
            pl.BlockSpec((NC, OBLK, NCLS), lambda i: (0, i, 0)),
            pl.BlockSpec((NC, OBLK, 16), lambda i: (0, i, 0)),
            pl.BlockSpec((1, NCLS), lambda i: (0, 0)),
        ],
        out_specs=pl.BlockSpec((OBLK, NCLS), lambda i: (i, 0)),
        out_shape=jax.ShapeDtypeStruct((N, NCLS), jnp.float32),
    )(agg2, id_p, b2.reshape(1, NCLS))

    return out
